# Initial kernel scaffold; baseline (speedup 1.0000x reference)
import jax
import jax.numpy as jnp
from jax.experimental import pallas as pl


def kernel(src, tgt, lengths, context_mask, context_lengthes, embedding, W_sent, W_ctx, W_dec):
    raise NotImplementedError("write your pallas kernel here")


# speedup vs baseline: 1.0900x; 1.0900x over previous
import functools

import jax
import jax.numpy as jnp
from jax import lax
from jax.experimental import pallas as pl
from jax.experimental.pallas import tpu as pltpu
from jax.experimental.pallas import tpu_sc as plsc

C = 32
_NW = 32
_CHUNK = 128


def _sc_gather(idx2d, table, rows_per_worker):
    nw, _, chunk = idx2d.shape
    n_chunks = nw * idx2d.shape[1]
    d = table.shape[1]
    chunks_per_w = rows_per_worker // chunk
    mesh = plsc.VectorSubcoreMesh(core_axis_name="c", subcore_axis_name="s")

    @functools.partial(
        pl.kernel,
        out_type=jax.ShapeDtypeStruct((n_chunks * chunk, d), jnp.float32),
        mesh=mesh,
        scratch_types=[
            pltpu.VMEM((chunks_per_w, chunk), jnp.int32),
            pltpu.VMEM((chunk, d), jnp.float32),
            pltpu.SemaphoreType.DMA,
        ],
    )
    def gather_kernel(idx_hbm, table_hbm, out_hbm, idx_v, rows_v, sem):
        nc = 2
        wid = lax.axis_index("s") * nc + lax.axis_index("c")
        pltpu.sync_copy(idx_hbm.at[wid], idx_v)
        for k in range(chunks_per_w):
            pltpu.async_copy(table_hbm.at[idx_v.at[k]], rows_v, sem).wait()
            pltpu.sync_copy(
                rows_v, out_hbm.at[pl.ds(wid * rows_per_worker + k * chunk, chunk)]
            )

    return gather_kernel(idx2d, table)


def _tc_body(lens_ref, clens_ref, emb_ref, dec_ref, cmask_ref, ws_ref, wc_ref,
             wd_ref, out_ref, sattn_ref, cattn_ref):
    b = pl.program_id(0)
    s = emb_ref.shape[1]
    td = dec_ref.shape[1]
    length = lens_ref[b]
    clen = clens_ref[b]

    emb = emb_ref[0]
    h = jnp.tanh(jnp.dot(emb, ws_ref[...], preferred_element_type=jnp.float32))

    iota_s_col = lax.broadcasted_iota(jnp.int32, (s, 1), 0)
    valid_col = (iota_s_col < length).astype(jnp.float32)
    sm = h * valid_col

    cmask = cmask_ref[0]
    iota_c = lax.broadcasted_iota(jnp.int32, (C, s), 0)
    iota_s_row = lax.broadcasted_iota(jnp.int32, (C, s), 1)
    validf = (iota_s_row < length).astype(jnp.float32)
    le = (cmask <= iota_c).astype(jnp.float32) * validf
    eq = (cmask == iota_c).astype(jnp.float32) * validf
    csum = jnp.sum(le, axis=1, keepdims=True)
    cnt = jnp.sum(eq, axis=1, keepdims=True)
    ends = jnp.clip(csum - 1.0, 0.0, float(s - 1)).astype(jnp.int32)
    onehot = (iota_s_row == ends)
    ci = jnp.dot(onehot.astype(jnp.float32), h,
                 preferred_element_type=jnp.float32)
    ci = ci * (cnt > 0.0).astype(jnp.float32)
    cmb = jnp.tanh(jnp.dot(ci, wc_ref[...], preferred_element_type=jnp.float32))

    last = jnp.clip(clen - 1, 0, C - 1)
    iota_cc = lax.broadcasted_iota(jnp.int32, (1, C), 1)
    oh_last = (iota_cc == last).astype(jnp.float32)
    ctx_final = jnp.dot(oh_last, cmb, preferred_element_type=jnp.float32)

    dec = dec_ref[0]
    dec_h = jnp.tanh(
        jnp.dot(dec, wd_ref[...], preferred_element_type=jnp.float32) + ctx_final)

    scale = 1.0 / jnp.sqrt(jnp.float32(emb.shape[1]))
    s_scores = lax.dot_general(dec_h, sm, (((1,), (1,)), ((), ())),
                               preferred_element_type=jnp.float32) * scale
    valid_row = lax.broadcasted_iota(jnp.int32, (1, s), 1) < length
    s_scores = jnp.where(valid_row, s_scores, -1e9)
    s_max = jnp.max(s_scores, axis=-1, keepdims=True)
    s_exp = jnp.exp(s_scores - s_max)
    s_attn = s_exp / jnp.sum(s_exp, axis=-1, keepdims=True)
    s_ctx = jnp.dot(s_attn, sm, preferred_element_type=jnp.float32)

    c_scores = lax.dot_general(dec_h, cmb, (((1,), (1,)), ((), ())),
                               preferred_element_type=jnp.float32) * scale
    c_scores = jnp.where(iota_cc < clen, c_scores, -1e9)
    c_max = jnp.max(c_scores, axis=-1, keepdims=True)
    c_exp = jnp.exp(c_scores - c_max)
    c_attn = c_exp / jnp.sum(c_exp, axis=-1, keepdims=True)
    c_ctx = jnp.dot(c_attn, cmb, preferred_element_type=jnp.float32)

    out_ref[0] = jnp.tanh(dec_h + s_ctx + c_ctx)
    sattn_ref[0] = s_attn
    cattn_ref[0] = c_attn


def kernel(src, tgt, lengths, context_mask, context_lengthes, embedding,
           W_sent, W_ctx, W_dec):
    s, b = src.shape
    td = tgt.shape[0] - 1
    d = embedding.shape[1]

    idx_src = src.T.reshape(-1).astype(jnp.int32)
    idx_tgt = tgt[:td].T.reshape(-1).astype(jnp.int32)
    n = b * s + b * td
    rows_per_worker = -(-n // (_NW * _CHUNK)) * _CHUNK
    npad = _NW * rows_per_worker - n
    idx_all = jnp.concatenate(
        [idx_src, idx_tgt, jnp.zeros((npad,), jnp.int32)])
    idx2d = idx_all.reshape(_NW, -1, _CHUNK)
    gathered = _sc_gather(idx2d, embedding, rows_per_worker)
    emb = gathered[: b * s].reshape(b, s, d)
    dec_emb = gathered[b * s: b * s + b * td].reshape(b, td, d)
    cmask3 = context_mask.T.reshape(b, 1, s)

    grid_spec = pltpu.PrefetchScalarGridSpec(
        num_scalar_prefetch=2,
        grid=(b,),
        in_specs=[
            pl.BlockSpec((1, s, d), lambda i, *_: (i, 0, 0)),
            pl.BlockSpec((1, td, d), lambda i, *_: (i, 0, 0)),
            pl.BlockSpec((1, 1, s), lambda i, *_: (i, 0, 0)),
            pl.BlockSpec((d, d), lambda i, *_: (0, 0)),
            pl.BlockSpec((d, d), lambda i, *_: (0, 0)),
            pl.BlockSpec((d, d), lambda i, *_: (0, 0)),
        ],
        out_specs=[
            pl.BlockSpec((1, td, d), lambda i, *_: (i, 0, 0)),
            pl.BlockSpec((1, td, s), lambda i, *_: (i, 0, 0)),
            pl.BlockSpec((1, td, C), lambda i, *_: (i, 0, 0)),
        ],
    )
    dec_out, s_attn, c_attn = pl.pallas_call(
        _tc_body,
        grid_spec=grid_spec,
        out_shape=[
            jax.ShapeDtypeStruct((b, td, d), jnp.float32),
            jax.ShapeDtypeStruct((b, td, s), jnp.float32),
            jax.ShapeDtypeStruct((b, td, C), jnp.float32),
        ],
    )(lengths.astype(jnp.int32), context_lengthes.astype(jnp.int32),
      emb, dec_emb, cmask3, W_sent, W_ctx, W_dec)

    return (jnp.transpose(dec_out, (1, 0, 2)),
            jnp.transpose(s_attn, (1, 0, 2)),
            jnp.transpose(c_attn, (1, 0, 2)))


# speedup vs baseline: 1.1606x; 1.0647x over previous
import functools

import jax
import jax.numpy as jnp
from jax import lax
from jax.experimental import pallas as pl
from jax.experimental.pallas import tpu as pltpu
from jax.experimental.pallas import tpu_sc as plsc

C = 32
_NW = 32
_CHUNK = 128


def _sc_gather(idx2d, table, rows_per_worker):
    nw, _, chunk = idx2d.shape
    n_chunks = nw * idx2d.shape[1]
    d = table.shape[1]
    chunks_per_w = rows_per_worker // chunk
    group = 3
    n_groups = -(-chunks_per_w // group)
    mesh = plsc.VectorSubcoreMesh(core_axis_name="c", subcore_axis_name="s")

    @functools.partial(
        pl.kernel,
        out_type=jax.ShapeDtypeStruct((n_chunks * chunk, d), jnp.float32),
        mesh=mesh,
        scratch_types=[
            pltpu.VMEM((chunks_per_w, chunk), jnp.int32),
            pltpu.VMEM((group * chunk, d), jnp.float32),
            pltpu.VMEM((group * chunk, d), jnp.float32),
            pltpu.SemaphoreType.DMA,
            pltpu.SemaphoreType.DMA,
            pltpu.SemaphoreType.DMA,
            pltpu.SemaphoreType.DMA,
        ],
    )
    def gather_kernel(idx_hbm, table_hbm, out_hbm, idx_v, buf_a, buf_b,
                      gsem_a, gsem_b, ssem_a, ssem_b):
        nc = 2
        wid = lax.axis_index("s") * nc + lax.axis_index("c")
        bufs = (buf_a, buf_b)
        gsems = (gsem_a, gsem_b)
        ssems = (ssem_a, ssem_b)
        pltpu.sync_copy(idx_hbm.at[wid], idx_v)

        def chunks_of(g):
            return range(g * group, min((g + 1) * group, chunks_per_w))

        def fire(g):
            buf, gsem = bufs[g % 2], gsems[g % 2]
            cps = []
            for j, k in enumerate(chunks_of(g)):
                cps.append(pltpu.async_copy(
                    table_hbm.at[idx_v.at[k]],
                    buf.at[pl.ds(j * chunk, chunk)], gsem))
            return cps

        gathers = {0: fire(0)}
        stores = {}
        for g in range(n_groups):
            if g + 1 < n_groups:
                if g - 1 >= 0:
                    stores.pop(g - 1).wait()
                gathers[g + 1] = fire(g + 1)
            for cp in gathers.pop(g):
                cp.wait()
            ks = list(chunks_of(g))
            stores[g] = pltpu.async_copy(
                bufs[g % 2].at[pl.ds(0, len(ks) * chunk)],
                out_hbm.at[pl.ds(wid * rows_per_worker + ks[0] * chunk,
                                 len(ks) * chunk)],
                ssems[g % 2])
        for g in sorted(stores):
            stores.pop(g).wait()

    return gather_kernel(idx2d, table)


def _tc_body(lens_ref, clens_ref, emb_ref, dec_ref, cmask_ref, ws_ref, wc_ref,
             wd_ref, out_ref, sattn_ref, cattn_ref):
    b = pl.program_id(0)
    s = emb_ref.shape[1]
    td = dec_ref.shape[1]
    length = lens_ref[b]
    clen = clens_ref[b]

    emb = emb_ref[0]
    h = jnp.tanh(jnp.dot(emb, ws_ref[...], preferred_element_type=jnp.float32))

    iota_s_col = lax.broadcasted_iota(jnp.int32, (s, 1), 0)
    valid_col = (iota_s_col < length).astype(jnp.float32)
    sm = h * valid_col

    cmask = cmask_ref[0]
    iota_c = lax.broadcasted_iota(jnp.int32, (C, s), 0)
    iota_s_row = lax.broadcasted_iota(jnp.int32, (C, s), 1)
    validf = (iota_s_row < length).astype(jnp.float32)
    le = (cmask <= iota_c).astype(jnp.float32) * validf
    eq = (cmask == iota_c).astype(jnp.float32) * validf
    csum = jnp.sum(le, axis=1, keepdims=True)
    cnt = jnp.sum(eq, axis=1, keepdims=True)
    ends = jnp.clip(csum - 1.0, 0.0, float(s - 1)).astype(jnp.int32)
    onehot = (iota_s_row == ends)
    ci = jnp.dot(onehot.astype(jnp.float32), h,
                 preferred_element_type=jnp.float32)
    ci = ci * (cnt > 0.0).astype(jnp.float32)
    cmb = jnp.tanh(jnp.dot(ci, wc_ref[...], preferred_element_type=jnp.float32))

    last = jnp.clip(clen - 1, 0, C - 1)
    iota_cc = lax.broadcasted_iota(jnp.int32, (1, C), 1)
    oh_last = (iota_cc == last).astype(jnp.float32)
    ctx_final = jnp.dot(oh_last, cmb, preferred_element_type=jnp.float32)

    dec = dec_ref[0]
    dec_h = jnp.tanh(
        jnp.dot(dec, wd_ref[...], preferred_element_type=jnp.float32) + ctx_final)

    scale = 1.0 / jnp.sqrt(jnp.float32(emb.shape[1]))
    s_scores = lax.dot_general(dec_h, sm, (((1,), (1,)), ((), ())),
                               preferred_element_type=jnp.float32) * scale
    valid_row = lax.broadcasted_iota(jnp.int32, (1, s), 1) < length
    s_scores = jnp.where(valid_row, s_scores, -1e9)
    s_max = jnp.max(s_scores, axis=-1, keepdims=True)
    s_exp = jnp.exp(s_scores - s_max)
    s_attn = s_exp / jnp.sum(s_exp, axis=-1, keepdims=True)
    s_ctx = jnp.dot(s_attn, sm, preferred_element_type=jnp.float32)

    c_scores = lax.dot_general(dec_h, cmb, (((1,), (1,)), ((), ())),
                               preferred_element_type=jnp.float32) * scale
    c_scores = jnp.where(iota_cc < clen, c_scores, -1e9)
    c_max = jnp.max(c_scores, axis=-1, keepdims=True)
    c_exp = jnp.exp(c_scores - c_max)
    c_attn = c_exp / jnp.sum(c_exp, axis=-1, keepdims=True)
    c_ctx = jnp.dot(c_attn, cmb, preferred_element_type=jnp.float32)

    out_ref[0] = jnp.tanh(dec_h + s_ctx + c_ctx)
    sattn_ref[0] = s_attn
    cattn_ref[0] = c_attn


def kernel(src, tgt, lengths, context_mask, context_lengthes, embedding,
           W_sent, W_ctx, W_dec):
    s, b = src.shape
    td = tgt.shape[0] - 1
    d = embedding.shape[1]

    idx_src = src.T.reshape(-1).astype(jnp.int32)
    idx_tgt = tgt[:td].T.reshape(-1).astype(jnp.int32)
    n = b * s + b * td
    rows_per_worker = -(-n // (_NW * _CHUNK)) * _CHUNK
    npad = _NW * rows_per_worker - n
    idx_all = jnp.concatenate(
        [idx_src, idx_tgt, jnp.zeros((npad,), jnp.int32)])
    idx2d = idx_all.reshape(_NW, -1, _CHUNK)
    gathered = _sc_gather(idx2d, embedding, rows_per_worker)
    emb = gathered[: b * s].reshape(b, s, d)
    dec_emb = gathered[b * s: b * s + b * td].reshape(b, td, d)
    cmask3 = context_mask.T.reshape(b, 1, s)

    grid_spec = pltpu.PrefetchScalarGridSpec(
        num_scalar_prefetch=2,
        grid=(b,),
        in_specs=[
            pl.BlockSpec((1, s, d), lambda i, *_: (i, 0, 0)),
            pl.BlockSpec((1, td, d), lambda i, *_: (i, 0, 0)),
            pl.BlockSpec((1, 1, s), lambda i, *_: (i, 0, 0)),
            pl.BlockSpec((d, d), lambda i, *_: (0, 0)),
            pl.BlockSpec((d, d), lambda i, *_: (0, 0)),
            pl.BlockSpec((d, d), lambda i, *_: (0, 0)),
        ],
        out_specs=[
            pl.BlockSpec((1, td, d), lambda i, *_: (i, 0, 0)),
            pl.BlockSpec((1, td, s), lambda i, *_: (i, 0, 0)),
            pl.BlockSpec((1, td, C), lambda i, *_: (i, 0, 0)),
        ],
    )
    dec_out, s_attn, c_attn = pl.pallas_call(
        _tc_body,
        grid_spec=grid_spec,
        out_shape=[
            jax.ShapeDtypeStruct((b, td, d), jnp.float32),
            jax.ShapeDtypeStruct((b, td, s), jnp.float32),
            jax.ShapeDtypeStruct((b, td, C), jnp.float32),
        ],
    )(lengths.astype(jnp.int32), context_lengthes.astype(jnp.int32),
      emb, dec_emb, cmask3, W_sent, W_ctx, W_dec)

    return (jnp.transpose(dec_out, (1, 0, 2)),
            jnp.transpose(s_attn, (1, 0, 2)),
            jnp.transpose(c_attn, (1, 0, 2)))


# speedup vs baseline: 2.2774x; 1.9623x over previous
import functools

import jax
import jax.numpy as jnp
from jax import lax
from jax.experimental import pallas as pl
from jax.experimental.pallas import tpu as pltpu
from jax.experimental.pallas import tpu_sc as plsc

C = 32
_NW = 32
_CHUNK = 128


def _sc_gather(idx_flat, table, rows_per_worker, n_groups):
    n = idx_flat.shape[0]
    d = table.shape[1]
    mesh = plsc.VectorSubcoreMesh(core_axis_name="c", subcore_axis_name="s")

    gchunk = rows_per_worker // n_groups

    @functools.partial(
        pl.kernel,
        out_type=jax.ShapeDtypeStruct((n, d), jnp.float32),
        mesh=mesh,
        scratch_types=[
            pltpu.VMEM((rows_per_worker,), jnp.int32),
            pltpu.VMEM((gchunk, d), jnp.float32),
            pltpu.VMEM((gchunk, d), jnp.float32),
            pltpu.SemaphoreType.DMA,
            pltpu.SemaphoreType.DMA,
            pltpu.SemaphoreType.DMA,
            pltpu.SemaphoreType.DMA,
        ],
    )
    def gather_kernel(idx_hbm, table_hbm, out_hbm, idx_v, buf_a, buf_b,
                      gsem_a, gsem_b, ssem_a, ssem_b):
        nc = 2
        wid = lax.axis_index("s") * nc + lax.axis_index("c")
        bufs = (buf_a, buf_b)
        gsems = (gsem_a, gsem_b)
        ssems = (ssem_a, ssem_b)
        pltpu.sync_copy(
            idx_hbm.at[pl.ds(wid * rows_per_worker, rows_per_worker)], idx_v)

        def fire(g):
            return pltpu.async_copy(
                table_hbm.at[idx_v.at[pl.ds(g * gchunk, gchunk)]],
                bufs[g % 2], gsems[g % 2])

        gathers = {0: fire(0)}
        stores = {}
        for g in range(n_groups):
            if g + 1 < n_groups:
                if g - 1 >= 0:
                    stores.pop(g - 1).wait()
                gathers[g + 1] = fire(g + 1)
            gathers.pop(g).wait()
            stores[g] = pltpu.async_copy(
                bufs[g % 2],
                out_hbm.at[pl.ds(wid * rows_per_worker + g * gchunk, gchunk)],
                ssems[g % 2])
        for g in sorted(stores):
            stores.pop(g).wait()

    return gather_kernel(idx_flat, table)


def _tc_body(lens_ref, clens_ref, emb_ref, dec_ref, cmask_ref, ws_ref, wc_ref,
             wd_ref, out_ref, sattn_ref, cattn_ref):
    b = pl.program_id(0)
    s = emb_ref.shape[1]
    td = dec_ref.shape[1]
    length = lens_ref[b]
    clen = clens_ref[b]

    emb = emb_ref[0]
    h = jnp.tanh(jnp.dot(emb, ws_ref[...], preferred_element_type=jnp.float32))

    iota_s_col = lax.broadcasted_iota(jnp.int32, (s, 1), 0)
    valid_col = (iota_s_col < length).astype(jnp.float32)
    sm = h * valid_col

    cmask = cmask_ref[0]
    iota_c = lax.broadcasted_iota(jnp.int32, (C, s), 0)
    iota_s_row = lax.broadcasted_iota(jnp.int32, (C, s), 1)
    validf = (iota_s_row < length).astype(jnp.float32)
    le = (cmask <= iota_c).astype(jnp.float32) * validf
    eq = (cmask == iota_c).astype(jnp.float32) * validf
    csum = jnp.sum(le, axis=1, keepdims=True)
    cnt = jnp.sum(eq, axis=1, keepdims=True)
    ends = jnp.clip(csum - 1.0, 0.0, float(s - 1)).astype(jnp.int32)
    onehot = (iota_s_row == ends)
    ci = jnp.dot(onehot.astype(jnp.float32), h,
                 preferred_element_type=jnp.float32)
    ci = ci * (cnt > 0.0).astype(jnp.float32)
    cmb = jnp.tanh(jnp.dot(ci, wc_ref[...], preferred_element_type=jnp.float32))

    last = jnp.clip(clen - 1, 0, C - 1)
    iota_cc = lax.broadcasted_iota(jnp.int32, (1, C), 1)
    oh_last = (iota_cc == last).astype(jnp.float32)
    ctx_final = jnp.dot(oh_last, cmb, preferred_element_type=jnp.float32)

    dec = dec_ref[0]
    dec_h = jnp.tanh(
        jnp.dot(dec, wd_ref[...], preferred_element_type=jnp.float32) + ctx_final)

    scale = 1.0 / jnp.sqrt(jnp.float32(emb.shape[1]))
    s_scores = lax.dot_general(dec_h, sm, (((1,), (1,)), ((), ())),
                               preferred_element_type=jnp.float32) * scale
    valid_row = lax.broadcasted_iota(jnp.int32, (1, s), 1) < length
    s_scores = jnp.where(valid_row, s_scores, -1e9)
    s_max = jnp.max(s_scores, axis=-1, keepdims=True)
    s_exp = jnp.exp(s_scores - s_max)
    s_attn = s_exp / jnp.sum(s_exp, axis=-1, keepdims=True)
    s_ctx = jnp.dot(s_attn, sm, preferred_element_type=jnp.float32)

    c_scores = lax.dot_general(dec_h, cmb, (((1,), (1,)), ((), ())),
                               preferred_element_type=jnp.float32) * scale
    c_scores = jnp.where(iota_cc < clen, c_scores, -1e9)
    c_max = jnp.max(c_scores, axis=-1, keepdims=True)
    c_exp = jnp.exp(c_scores - c_max)
    c_attn = c_exp / jnp.sum(c_exp, axis=-1, keepdims=True)
    c_ctx = jnp.dot(c_attn, cmb, preferred_element_type=jnp.float32)

    out_ref[0] = jnp.tanh(dec_h + s_ctx + c_ctx)
    sattn_ref[0] = s_attn
    cattn_ref[0] = c_attn


def kernel(src, tgt, lengths, context_mask, context_lengthes, embedding,
           W_sent, W_ctx, W_dec):
    s, b = src.shape
    td = tgt.shape[0] - 1
    d = embedding.shape[1]

    P = 4
    bp = b // P
    lengths32 = lengths.astype(jnp.int32)
    clens32 = context_lengthes.astype(jnp.int32)
    cmask_t = context_mask.T

    grid_spec = pltpu.PrefetchScalarGridSpec(
        num_scalar_prefetch=2,
        grid=(bp,),
        in_specs=[
            pl.BlockSpec((1, s, d), lambda i, *_: (i, 0, 0)),
            pl.BlockSpec((1, td, d), lambda i, *_: (i, 0, 0)),
            pl.BlockSpec((1, 1, s), lambda i, *_: (i, 0, 0)),
            pl.BlockSpec((d, d), lambda i, *_: (0, 0)),
            pl.BlockSpec((d, d), lambda i, *_: (0, 0)),
            pl.BlockSpec((d, d), lambda i, *_: (0, 0)),
        ],
        out_specs=[
            pl.BlockSpec((1, td, d), lambda i, *_: (i, 0, 0)),
            pl.BlockSpec((1, td, s), lambda i, *_: (i, 0, 0)),
            pl.BlockSpec((1, td, C), lambda i, *_: (i, 0, 0)),
        ],
    )
    tc_call = functools.partial(
        pl.pallas_call,
        _tc_body,
        grid_spec=grid_spec,
        out_shape=[
            jax.ShapeDtypeStruct((bp, td, d), jnp.float32),
            jax.ShapeDtypeStruct((bp, td, s), jnp.float32),
            jax.ShapeDtypeStruct((bp, td, C), jnp.float32),
        ],
    )

    parts = []
    for p in range(P):
        sl = slice(p * bp, (p + 1) * bp)
        idx_p = jnp.concatenate([
            src[:, sl].T.reshape(-1).astype(jnp.int32),
            tgt[:td, sl].T.reshape(-1).astype(jnp.int32)])
        rows_pw = idx_p.shape[0] // _NW
        g = _sc_gather(idx_p, embedding, rows_pw, 1)
        emb_p = g[: bp * s].reshape(bp, s, d)
        dec_p = g[bp * s:].reshape(bp, td, d)
        cm_p = cmask_t[sl].reshape(bp, 1, s)
        parts.append(tc_call()(
            lengths32[sl], clens32[sl], emb_p, dec_p, cm_p,
            W_sent, W_ctx, W_dec))

    dec_out = jnp.concatenate([o[0] for o in parts], axis=0)
    s_attn = jnp.concatenate([o[1] for o in parts], axis=0)
    c_attn = jnp.concatenate([o[2] for o in parts], axis=0)
    return (jnp.transpose(dec_out, (1, 0, 2)),
            jnp.transpose(s_attn, (1, 0, 2)),
            jnp.transpose(c_attn, (1, 0, 2)))
